# Initial kernel scaffold; baseline (speedup 1.0000x reference)
import functools

import jax
import jax.numpy as jnp
from jax import lax
from jax.experimental import pallas as pl
from jax.experimental.pallas import tpu as pltpu
from jax.experimental.pallas import tpu_sc as plsc

NC = 2
NS = 16
GROUP = 128
GB = 8
EDGE_ALIGN = NS * GROUP * GB
NGRAPH = 128


def _round_up(n, m):
    return ((n + m - 1) // m) * m


@functools.lru_cache(maxsize=None)
def _segsum_kernel(n_src_pad, n_groups, n_dst_pad):
    PT_G = n_groups // NS
    NB = PT_G // GB
    R = n_dst_pad // NS
    nz_full, nz_tail = R // GROUP, R % GROUP
    mesh = plsc.VectorSubcoreMesh(
        core_axis_name="c", subcore_axis_name="s",
        num_cores=NC, num_subcores=NS)

    def body(table, srcg, dstg, out, s_idx, d_idx, rows, zrow, acc,
             sem_g, sem_s):
        c = lax.axis_index("c")
        s = lax.axis_index("s")
        tview = table.at[c]
        oview = out.at[c]

        zv = jnp.zeros((16,), jnp.float32)

        def zr(i, carry):
            zrow[i, pl.ds(0, 16)] = zv
            zrow[i, pl.ds(16, 16)] = zv
            return carry

        lax.fori_loop(0, GROUP, zr, 0)
        base = s * R
        for i in range(nz_full):
            pltpu.sync_copy(zrow, acc.at[pl.ds(base + i * GROUP, GROUP)])
        if nz_tail:
            pltpu.sync_copy(zrow.at[pl.ds(0, nz_tail)],
                            acc.at[pl.ds(base + nz_full * GROUP, nz_tail)])
        plsc.subcore_barrier()

        g0 = s * PT_G

        def batch(bi, carry):
            gbase = g0 + bi * GB
            pltpu.sync_copy(srcg.at[pl.ds(gbase, GB)], s_idx)
            pltpu.sync_copy(dstg.at[pl.ds(gbase, GB)], d_idx)
            gd = [pltpu.async_copy(tview.at[s_idx.at[j]], rows.at[j], sem_g)
                  for j in range(GB)]
            for d in gd:
                d.wait()
            sd = [pltpu.async_copy(rows.at[j], acc.at[d_idx.at[j]], sem_s,
                                   add=True)
                  for j in range(GB)]
            for d in sd:
                d.wait()
            return carry

        lax.fori_loop(0, NB, batch, 0)
        plsc.subcore_barrier()
        pltpu.sync_copy(acc.at[pl.ds(base, R)], oview.at[pl.ds(base, R)])

    return pl.kernel(
        body,
        out_type=jax.ShapeDtypeStruct((NC, n_dst_pad, 32), jnp.float32),
        mesh=mesh,
        scratch_types=[
            pltpu.VMEM((GB, GROUP), jnp.int32),
            pltpu.VMEM((GB, GROUP), jnp.int32),
            pltpu.VMEM((GB, GROUP, 32), jnp.float32),
            pltpu.VMEM((GROUP, 32), jnp.float32),
            pltpu.VMEM_SHARED((n_dst_pad, 32), jnp.float32),
            pltpu.SemaphoreType.DMA,
            pltpu.SemaphoreType.DMA,
        ],
    )


def _seg(table_split, srcg, dstg, n_dst_pad):
    k = _segsum_kernel(table_split.shape[1], srcg.shape[0], n_dst_pad)
    return k(table_split, srcg, dstg)


def _pad_idx(src, dst, pad_src, pad_dst):
    n = src.shape[0]
    n_pad = _round_up(n, EDGE_ALIGN)
    pad = n_pad - n
    src = jnp.concatenate(
        [src.astype(jnp.int32), jnp.full((pad,), pad_src, jnp.int32)])
    dst = jnp.concatenate(
        [dst.astype(jnp.int32), jnp.full((pad,), pad_dst, jnp.int32)])
    return (src.reshape(n_pad // GROUP, GROUP),
            dst.reshape(n_pad // GROUP, GROUP))


def _mm(a_split, m_split, w, b, mode):
    n_pad = a_split.shape[1]
    nblk = n_pad // 128
    w = w.astype(jnp.float32)
    b2d = b.reshape(1, -1).astype(jnp.float32)

    def body(a_ref, m_ref, w_ref, b_ref, *outs):
        xx = jnp.concatenate([a_ref[0], a_ref[1], m_ref[0], m_ref[1]],
                             axis=-1)
        y = jnp.dot(xx, w_ref[...], preferred_element_type=jnp.float32)
        y = y + b_ref[...]
        if mode == "relu":
            y = jnp.maximum(y, 0.0)
        outs[0][0] = y[:, :32]
        outs[0][1] = y[:, 32:]
        if mode == "both":
            r = jnp.maximum(y, 0.0)
            outs[1][0] = r[:, :32]
            outs[1][1] = r[:, 32:]

    split_spec = pl.BlockSpec((2, 128, 32), lambda i: (0, i, 0))
    out_sds = jax.ShapeDtypeStruct((2, n_pad, 32), jnp.float32)
    n_out = 2 if mode == "both" else 1
    return pl.pallas_call(
        body,
        grid=(nblk,),
        in_specs=[
            split_spec,
            split_spec,
            pl.BlockSpec((128, 64), lambda i: (0, 0)),
            pl.BlockSpec((1, 64), lambda i: (0, 0)),
        ],
        out_specs=[split_spec] * n_out,
        out_shape=[out_sds] * n_out,
    )(a_split, m_split, w, b2d)


def _head(xg_split, eg_split, w1, b1, w2, b2):
    def body(xg_ref, eg_ref, w1_ref, b1_ref, w2_ref, b2_ref, out_ref):
        xx = jnp.concatenate(
            [xg_ref[0], xg_ref[1], eg_ref[0], eg_ref[1]], axis=-1)
        h = jnp.dot(xx, w1_ref[...], preferred_element_type=jnp.float32)
        h = jnp.maximum(h + b1_ref[...], 0.0)
        out_ref[...] = (jnp.dot(h, w2_ref[...],
                                preferred_element_type=jnp.float32)
                        + b2_ref[...])

    return pl.pallas_call(
        body,
        grid=(1,),
        in_specs=[
            pl.BlockSpec((2, 128, 32), lambda i: (0, 0, 0)),
            pl.BlockSpec((2, 128, 32), lambda i: (0, 0, 0)),
            pl.BlockSpec((128, 128), lambda i: (0, 0)),
            pl.BlockSpec((1, 128), lambda i: (0, 0)),
            pl.BlockSpec((128, 1), lambda i: (0, 0)),
            pl.BlockSpec((1, 1), lambda i: (0, 0)),
        ],
        out_specs=pl.BlockSpec((128, 1), lambda i: (0, 0)),
        out_shape=jax.ShapeDtypeStruct((128, 1), jnp.float32),
    )(xg_split, eg_split, w1.astype(jnp.float32),
      b1.reshape(1, -1).astype(jnp.float32), w2.astype(jnp.float32),
      b2.reshape(1, -1).astype(jnp.float32))


def kernel(x, edge_attr, edge_index0, edge_index1, batch, e_batch,
           atom_table, hbond_table, W_e, b_e, W_v, b_v, W1, b1, W2, b2):
    nv = x.shape[0]
    nhe = edge_attr.shape[0]
    nvp = _round_up(nv + 1, 128)
    nhep = _round_up(nhe + 1, 128)
    ngp = _round_up(NGRAPH + 1, 128)

    atom_s = atom_table.astype(jnp.float32).reshape(-1, 2, 32).transpose(1, 0, 2)
    hbond_s = hbond_table.astype(jnp.float32).reshape(-1, 2, 32).transpose(1, 0, 2)

    iota_v = jnp.arange(nv, dtype=jnp.int32)
    iota_e = jnp.arange(nhe, dtype=jnp.int32)
    xs, xd = _pad_idx(x, iota_v, 0, nv)
    es, ed = _pad_idx(edge_attr, iota_e, 0, nhe)
    e0g, e1g = _pad_idx(edge_index0, edge_index1, nv, nhe)
    pvs, pvd = _pad_idx(iota_v, batch, 0, NGRAPH)
    pes, ped = _pad_idx(iota_e, e_batch, 0, NGRAPH)

    xv = _seg(atom_s, xs, xd, nvp)
    ev = _seg(hbond_s, es, ed, nhep)

    for layer in range(3):
        m_e = _seg(xv, e0g, e1g, nhep)
        if layer < 2:
            ev_raw, ev_next = _mm(ev, m_e, W_e, b_e, "both")
        else:
            (ev_raw,) = _mm(ev, m_e, W_e, b_e, "raw")
            ev_next = ev_raw
        m_v = _seg(ev_raw, e1g, e0g, nvp)
        (xv,) = _mm(xv, m_v, W_v, b_v, "relu" if layer < 2 else "raw")
        ev = ev_next

    xg = _seg(xv, pvs, pvd, ngp)
    eg = _seg(ev, pes, ped, ngp)
    out = _head(xg, eg, W1, b1, W2, b2)
    return out.reshape(-1)


# speedup vs baseline: 3.8091x; 3.8091x over previous
import functools

import jax
import jax.numpy as jnp
from jax import lax
from jax.experimental import pallas as pl
from jax.experimental.pallas import tpu as pltpu
from jax.experimental.pallas import tpu_sc as plsc

NC = 2
NS = 16
NQ = 4
QW = 16
GROUP = 128
GB = 8
EDGE_ALIGN = NS * GROUP * GB
NGRAPH = 128


def _round_up(n, m):
    return ((n + m - 1) // m) * m


@functools.lru_cache(maxsize=None)
def _segsum_kernel(n_src_pad, n_groups, n_dst_pad):
    PT_G = n_groups // NS
    NB = PT_G // GB
    R = n_dst_pad // NS
    nz_full, nz_tail = R // GROUP, R % GROUP
    mesh = plsc.VectorSubcoreMesh(
        core_axis_name="c", subcore_axis_name="s",
        num_cores=NC, num_subcores=NS)

    def body(table, srcg, dstg, out, s_idx, d_idx, rows, zrow, acc,
             sem_g, sem_s):
        c = lax.axis_index("c")
        s = lax.axis_index("s")
        base = s * R
        g0 = s * PT_G
        zv = jnp.zeros((QW,), jnp.float32)

        def zr(i, carry):
            zrow[i, pl.ds(0, QW)] = zv
            return carry

        lax.fori_loop(0, GROUP, zr, 0)

        for p in range(2):
            q = 2 * p + c
            tview = table.at[q]
            oview = out.at[q]

            for i in range(nz_full):
                pltpu.sync_copy(zrow, acc.at[pl.ds(base + i * GROUP, GROUP)])
            if nz_tail:
                pltpu.sync_copy(
                    zrow.at[pl.ds(0, nz_tail)],
                    acc.at[pl.ds(base + nz_full * GROUP, nz_tail)])
            plsc.subcore_barrier()

            def batch(bi, carry):
                gbase = g0 + bi * GB
                pltpu.sync_copy(srcg.at[pl.ds(gbase, GB)], s_idx)
                pltpu.sync_copy(dstg.at[pl.ds(gbase, GB)], d_idx)
                gd = [pltpu.async_copy(tview.at[s_idx.at[j]], rows.at[j],
                                       sem_g)
                      for j in range(GB)]
                for d in gd:
                    d.wait()
                sd = [pltpu.async_copy(rows.at[j], acc.at[d_idx.at[j]],
                                       sem_s, add=True)
                      for j in range(GB)]
                for d in sd:
                    d.wait()
                return carry

            lax.fori_loop(0, NB, batch, 0)
            plsc.subcore_barrier()
            pltpu.sync_copy(acc.at[pl.ds(base, R)], oview.at[pl.ds(base, R)])
            plsc.subcore_barrier()

    return pl.kernel(
        body,
        out_type=jax.ShapeDtypeStruct((NQ, n_dst_pad, QW), jnp.float32),
        mesh=mesh,
        scratch_types=[
            pltpu.VMEM((GB, GROUP), jnp.int32),
            pltpu.VMEM((GB, GROUP), jnp.int32),
            pltpu.VMEM((GB, GROUP, QW), jnp.float32),
            pltpu.VMEM((GROUP, QW), jnp.float32),
            pltpu.VMEM_SHARED((n_dst_pad, QW), jnp.float32),
            pltpu.SemaphoreType.DMA,
            pltpu.SemaphoreType.DMA,
        ],
        compiler_params=pltpu.CompilerParams(use_tc_tiling_on_sc=False),
    )


def _seg(table_split, srcg, dstg, n_dst_pad):
    k = _segsum_kernel(table_split.shape[1], srcg.shape[0], n_dst_pad)
    return k(table_split, srcg, dstg)


def _pad_idx(src, dst, pad_src, pad_dst):
    n = src.shape[0]
    n_pad = _round_up(n, EDGE_ALIGN)
    pad = n_pad - n
    src = jnp.concatenate(
        [src.astype(jnp.int32), jnp.full((pad,), pad_src, jnp.int32)])
    dst = jnp.concatenate(
        [dst.astype(jnp.int32), jnp.full((pad,), pad_dst, jnp.int32)])
    return (src.reshape(n_pad // GROUP, GROUP),
            dst.reshape(n_pad // GROUP, GROUP))


def _to_split(t):
    return t.astype(jnp.float32).reshape(-1, NQ, QW).transpose(1, 0, 2)


def _mm(a_split, m_split, w, b, mode):
    n_pad = a_split.shape[1]
    nblk = n_pad // 128
    w = w.astype(jnp.float32)
    b2d = b.reshape(1, -1).astype(jnp.float32)

    def body(a_ref, m_ref, w_ref, b_ref, *outs):
        xx = jnp.concatenate(
            [a_ref[0], a_ref[1], a_ref[2], a_ref[3],
             m_ref[0], m_ref[1], m_ref[2], m_ref[3]], axis=-1)
        y = jnp.dot(xx, w_ref[...], preferred_element_type=jnp.float32)
        y = y + b_ref[...]
        if mode == "relu":
            y = jnp.maximum(y, 0.0)
        for qq in range(NQ):
            outs[0][qq] = y[:, qq * QW:(qq + 1) * QW]
        if mode == "both":
            r = jnp.maximum(y, 0.0)
            for qq in range(NQ):
                outs[1][qq] = r[:, qq * QW:(qq + 1) * QW]

    split_spec = pl.BlockSpec((NQ, 128, QW), lambda i: (0, i, 0))
    out_sds = jax.ShapeDtypeStruct((NQ, n_pad, QW), jnp.float32)
    n_out = 2 if mode == "both" else 1
    return pl.pallas_call(
        body,
        grid=(nblk,),
        in_specs=[
            split_spec,
            split_spec,
            pl.BlockSpec((128, 64), lambda i: (0, 0)),
            pl.BlockSpec((1, 64), lambda i: (0, 0)),
        ],
        out_specs=[split_spec] * n_out,
        out_shape=[out_sds] * n_out,
    )(a_split, m_split, w, b2d)


def _head(xg_split, eg_split, w1, b1, w2, b2):
    def body(xg_ref, eg_ref, w1_ref, b1_ref, w2_ref, b2_ref, out_ref):
        xx = jnp.concatenate(
            [xg_ref[0], xg_ref[1], xg_ref[2], xg_ref[3],
             eg_ref[0], eg_ref[1], eg_ref[2], eg_ref[3]], axis=-1)
        h = jnp.dot(xx, w1_ref[...], preferred_element_type=jnp.float32)
        h = jnp.maximum(h + b1_ref[...], 0.0)
        out_ref[...] = (jnp.dot(h, w2_ref[...],
                                preferred_element_type=jnp.float32)
                        + b2_ref[...])

    return pl.pallas_call(
        body,
        grid=(1,),
        in_specs=[
            pl.BlockSpec((NQ, 128, QW), lambda i: (0, 0, 0)),
            pl.BlockSpec((NQ, 128, QW), lambda i: (0, 0, 0)),
            pl.BlockSpec((128, 128), lambda i: (0, 0)),
            pl.BlockSpec((1, 128), lambda i: (0, 0)),
            pl.BlockSpec((128, 1), lambda i: (0, 0)),
            pl.BlockSpec((1, 1), lambda i: (0, 0)),
        ],
        out_specs=pl.BlockSpec((128, 1), lambda i: (0, 0)),
        out_shape=jax.ShapeDtypeStruct((128, 1), jnp.float32),
    )(xg_split, eg_split, w1.astype(jnp.float32),
      b1.reshape(1, -1).astype(jnp.float32), w2.astype(jnp.float32),
      b2.reshape(1, -1).astype(jnp.float32))


def kernel(x, edge_attr, edge_index0, edge_index1, batch, e_batch,
           atom_table, hbond_table, W_e, b_e, W_v, b_v, W1, b1, W2, b2):
    nv = x.shape[0]
    nhe = edge_attr.shape[0]
    nvp = _round_up(nv + 1, 128)
    nhep = _round_up(nhe + 1, 128)
    ngp = _round_up(NGRAPH + 1, 128)

    atom_s = _to_split(atom_table)
    hbond_s = _to_split(hbond_table)

    iota_v = jnp.arange(nv, dtype=jnp.int32)
    iota_e = jnp.arange(nhe, dtype=jnp.int32)
    xs, xd = _pad_idx(x, iota_v, 0, nv)
    es, ed = _pad_idx(edge_attr, iota_e, 0, nhe)
    e0g, e1g = _pad_idx(edge_index0, edge_index1, nv, nhe)
    pvs, pvd = _pad_idx(iota_v, batch, 0, NGRAPH)
    pes, ped = _pad_idx(iota_e, e_batch, 0, NGRAPH)

    xv = _seg(atom_s, xs, xd, nvp)
    ev = _seg(hbond_s, es, ed, nhep)

    for layer in range(3):
        m_e = _seg(xv, e0g, e1g, nhep)
        if layer < 2:
            ev_raw, ev_next = _mm(ev, m_e, W_e, b_e, "both")
        else:
            (ev_raw,) = _mm(ev, m_e, W_e, b_e, "raw")
            ev_next = ev_raw
        m_v = _seg(ev_raw, e1g, e0g, nvp)
        (xv,) = _mm(xv, m_v, W_v, b_v, "relu" if layer < 2 else "raw")
        ev = ev_next

    xg = _seg(xv, pvs, pvd, ngp)
    eg = _seg(ev, pes, ped, ngp)
    out = _head(xg, eg, W1, b1, W2, b2)
    return out.reshape(-1)


# speedup vs baseline: 4.1505x; 1.0896x over previous
import functools

import jax
import jax.numpy as jnp
from jax import lax
from jax.experimental import pallas as pl
from jax.experimental.pallas import tpu as pltpu
from jax.experimental.pallas import tpu_sc as plsc

NC = 2
NS = 16
GROUP = 128
GB = 8
EDGE_ALIGN = NS * GROUP * GB * 2
NGRAPH = 128


def _round_up(n, m):
    return ((n + m - 1) // m) * m


@functools.lru_cache(maxsize=None)
def _segsum_kernel(n_src_pad, n_groups, n_dst_pad, w):
    PT_G = n_groups // NS
    NB = PT_G // GB
    R = n_dst_pad // NS
    nz_full, nz_tail = R // GROUP, R % GROUP
    n_split = 64 // w
    n_sub = n_split // NC
    mesh = plsc.VectorSubcoreMesh(
        core_axis_name="c", subcore_axis_name="s",
        num_cores=NC, num_subcores=NS)

    def body(table, srcg, dstg, out, s_idx0, s_idx1, d_idx0, d_idx1,
             rows0, rows1, zrow, acc, sg0, sg1, ss0, ss1):
        c = lax.axis_index("c")
        s = lax.axis_index("s")
        base = s * R
        g0 = s * PT_G
        s_idx = (s_idx0, s_idx1)
        d_idx = (d_idx0, d_idx1)
        rows = (rows0, rows1)
        sem_g = (sg0, sg1)
        sem_s = (ss0, ss1)
        zv = jnp.zeros((16,), jnp.float32)

        def zr(i, carry):
            for o in range(w // 16):
                zrow[i, pl.ds(16 * o, 16)] = zv
            return carry

        lax.fori_loop(0, GROUP, zr, 0)

        def load_idx(buf, b):
            gbase = g0 + b * GB
            pltpu.sync_copy(srcg.at[pl.ds(gbase, GB)], s_idx[buf])
            pltpu.sync_copy(dstg.at[pl.ds(gbase, GB)], d_idx[buf])

        for p in range(n_sub):
            q = NC * p + c
            tview = table.at[q]
            oview = out.at[q]

            for i in range(nz_full):
                pltpu.sync_copy(zrow, acc.at[pl.ds(base + i * GROUP, GROUP)])
            if nz_tail:
                pltpu.sync_copy(
                    zrow.at[pl.ds(0, nz_tail)],
                    acc.at[pl.ds(base + nz_full * GROUP, nz_tail)])
            plsc.subcore_barrier()

            def fire_gathers(buf):
                for j in range(GB):
                    pltpu.async_copy(tview.at[s_idx[buf].at[j]],
                                     rows[buf].at[j], sem_g[buf])

            for buf in (0, 1):
                load_idx(buf, buf)
                fire_gathers(buf)

            def pair(k, carry):
                for buf in (0, 1):
                    b = 2 * k + buf
                    for j in range(GB):
                        pltpu.make_async_copy(
                            tview.at[s_idx[buf].at[j]], rows[buf].at[j],
                            sem_g[buf]).wait()
                    sd = [pltpu.async_copy(rows[buf].at[j],
                                           acc.at[d_idx[buf].at[j]],
                                           sem_s[buf], add=True)
                          for j in range(GB)]
                    for d in sd:
                        d.wait()

                    @pl.when(b + 2 < NB)
                    def _():
                        load_idx(buf, b + 2)
                        fire_gathers(buf)
                return carry

            lax.fori_loop(0, NB // 2, pair, 0)
            plsc.subcore_barrier()
            pltpu.sync_copy(acc.at[pl.ds(base, R)], oview.at[pl.ds(base, R)])
            plsc.subcore_barrier()

    return pl.kernel(
        body,
        out_type=jax.ShapeDtypeStruct((n_split, n_dst_pad, w), jnp.float32),
        mesh=mesh,
        scratch_types=[
            pltpu.VMEM((GB, GROUP), jnp.int32),
            pltpu.VMEM((GB, GROUP), jnp.int32),
            pltpu.VMEM((GB, GROUP), jnp.int32),
            pltpu.VMEM((GB, GROUP), jnp.int32),
            pltpu.VMEM((GB, GROUP, w), jnp.float32),
            pltpu.VMEM((GB, GROUP, w), jnp.float32),
            pltpu.VMEM((GROUP, w), jnp.float32),
            pltpu.VMEM_SHARED((n_dst_pad, w), jnp.float32),
            pltpu.SemaphoreType.DMA,
            pltpu.SemaphoreType.DMA,
            pltpu.SemaphoreType.DMA,
            pltpu.SemaphoreType.DMA,
        ],
        compiler_params=pltpu.CompilerParams(use_tc_tiling_on_sc=False),
    )


def _seg(table_split, srcg, dstg, n_dst_pad):
    w = table_split.shape[2]
    k = _segsum_kernel(table_split.shape[1], srcg.shape[0], n_dst_pad, w)
    return k(table_split, srcg, dstg)


def _pad_idx(src, dst, pad_src, pad_dst):
    n = src.shape[0]
    n_pad = _round_up(n, EDGE_ALIGN)
    pad = n_pad - n
    src = jnp.concatenate(
        [src.astype(jnp.int32), jnp.full((pad,), pad_src, jnp.int32)])
    dst = jnp.concatenate(
        [dst.astype(jnp.int32), jnp.full((pad,), pad_dst, jnp.int32)])
    return (src.reshape(n_pad // GROUP, GROUP),
            dst.reshape(n_pad // GROUP, GROUP))


def _embed_tc(idx_pad, table, n_pad, n_split):
    nblk = n_pad // 128
    wp = 64 // n_split
    vocab = table.shape[0]
    t_pad = jnp.zeros((128, 64), jnp.float32).at[:vocab].set(
        table.astype(jnp.float32))
    idx3 = idx_pad.reshape(nblk, 1, 128)

    def body(idx_ref, t_ref, out_ref):
        iv = idx_ref[0, 0, :]
        oh = (iv[:, None]
              == lax.broadcasted_iota(jnp.int32, (128, 128), 1))
        y = jnp.dot(oh.astype(jnp.float32), t_ref[...],
                    preferred_element_type=jnp.float32)
        for q in range(n_split):
            out_ref[q] = y[:, q * wp:(q + 1) * wp]

    return pl.pallas_call(
        body,
        grid=(nblk,),
        in_specs=[
            pl.BlockSpec((1, 1, 128), lambda i: (i, 0, 0)),
            pl.BlockSpec((128, 64), lambda i: (0, 0)),
        ],
        out_specs=pl.BlockSpec((n_split, 128, wp), lambda i: (0, i, 0)),
        out_shape=jax.ShapeDtypeStruct((n_split, n_pad, wp), jnp.float32),
    )(idx3, t_pad)


def _mm(a_split, m_split, w, b, mode):
    n_pad = a_split.shape[1]
    nblk = n_pad // 128
    sa, wa = a_split.shape[0], a_split.shape[2]
    sm, wm = m_split.shape[0], m_split.shape[2]
    w = w.astype(jnp.float32)
    b2d = b.reshape(1, -1).astype(jnp.float32)

    def body(a_ref, m_ref, w_ref, b_ref, *outs):
        xx = jnp.concatenate(
            [a_ref[i] for i in range(sa)] + [m_ref[i] for i in range(sm)],
            axis=-1)
        y = jnp.dot(xx, w_ref[...], preferred_element_type=jnp.float32)
        y = y + b_ref[...]
        if mode == "relu":
            y = jnp.maximum(y, 0.0)
        for q in range(sa):
            outs[0][q] = y[:, q * wa:(q + 1) * wa]
        if mode == "both":
            r = jnp.maximum(y, 0.0)
            for q in range(sa):
                outs[1][q] = r[:, q * wa:(q + 1) * wa]

    a_spec = pl.BlockSpec((sa, 128, wa), lambda i: (0, i, 0))
    m_spec = pl.BlockSpec((sm, 128, wm), lambda i: (0, i, 0))
    out_sds = jax.ShapeDtypeStruct((sa, n_pad, wa), jnp.float32)
    n_out = 2 if mode == "both" else 1
    return pl.pallas_call(
        body,
        grid=(nblk,),
        in_specs=[
            a_spec,
            m_spec,
            pl.BlockSpec((128, 64), lambda i: (0, 0)),
            pl.BlockSpec((1, 64), lambda i: (0, 0)),
        ],
        out_specs=[a_spec] * n_out,
        out_shape=[out_sds] * n_out,
    )(a_split, m_split, w, b2d)


def _head(xg_split, eg_split, w1, b1, w2, b2):
    def body(xg_ref, eg_ref, w1_ref, b1_ref, w2_ref, b2_ref, out_ref):
        xx = jnp.concatenate(
            [xg_ref[0], xg_ref[1],
             eg_ref[0], eg_ref[1], eg_ref[2], eg_ref[3]], axis=-1)
        h = jnp.dot(xx, w1_ref[...], preferred_element_type=jnp.float32)
        h = jnp.maximum(h + b1_ref[...], 0.0)
        out_ref[...] = (jnp.dot(h, w2_ref[...],
                                preferred_element_type=jnp.float32)
                        + b2_ref[...])

    return pl.pallas_call(
        body,
        grid=(1,),
        in_specs=[
            pl.BlockSpec((2, 128, 32), lambda i: (0, 0, 0)),
            pl.BlockSpec((4, 128, 16), lambda i: (0, 0, 0)),
            pl.BlockSpec((128, 128), lambda i: (0, 0)),
            pl.BlockSpec((1, 128), lambda i: (0, 0)),
            pl.BlockSpec((128, 1), lambda i: (0, 0)),
            pl.BlockSpec((1, 1), lambda i: (0, 0)),
        ],
        out_specs=pl.BlockSpec((128, 1), lambda i: (0, 0)),
        out_shape=jax.ShapeDtypeStruct((128, 1), jnp.float32),
    )(xg_split, eg_split, w1.astype(jnp.float32),
      b1.reshape(1, -1).astype(jnp.float32), w2.astype(jnp.float32),
      b2.reshape(1, -1).astype(jnp.float32))


def kernel(x, edge_attr, edge_index0, edge_index1, batch, e_batch,
           atom_table, hbond_table, W_e, b_e, W_v, b_v, W1, b1, W2, b2):
    nv = x.shape[0]
    nhe = edge_attr.shape[0]
    nvp = _round_up(nv + 1, 128)
    nhep = _round_up(nhe + 1, 128)
    ngp = _round_up(NGRAPH + 1, 128)

    x_pad = jnp.concatenate(
        [x.astype(jnp.int32), jnp.zeros((nvp - nv,), jnp.int32)])
    ea_pad = jnp.concatenate(
        [edge_attr.astype(jnp.int32), jnp.zeros((nhep - nhe,), jnp.int32)])

    iota_v = jnp.arange(nv, dtype=jnp.int32)
    iota_e = jnp.arange(nhe, dtype=jnp.int32)
    e0g, e1g = _pad_idx(edge_index0, edge_index1, nv, nhe)
    pvs, pvd = _pad_idx(iota_v, batch, 0, NGRAPH)
    pes, ped = _pad_idx(iota_e, e_batch, 0, NGRAPH)

    xv = _embed_tc(x_pad, atom_table, nvp, 2)
    ev = _embed_tc(ea_pad, hbond_table, nhep, 4)

    for layer in range(3):
        m_e = _seg(xv, e0g, e1g, nhep)
        if layer < 2:
            ev_raw, ev_next = _mm(ev, m_e, W_e, b_e, "both")
        else:
            (ev_raw,) = _mm(ev, m_e, W_e, b_e, "raw")
            ev_next = ev_raw
        m_v = _seg(ev_raw, e1g, e0g, nvp)
        (xv,) = _mm(xv, m_v, W_v, b_v, "relu" if layer < 2 else "raw")
        ev = ev_next

    xg = _seg(xv, pvs, pvd, ngp)
    eg = _seg(ev, pes, ped, ngp)
    out = _head(xg, eg, W1, b1, W2, b2)
    return out.reshape(-1)
